# baseline (speedup 1.0000x reference)
import functools

import jax
import jax.numpy as jnp
from jax import lax
from jax.experimental import pallas as pl
from jax.experimental.pallas import tpu as pltpu
from jax.experimental.pallas import tpu_sc as plsc

_THR = 0.3
_INTERPRET = False
_B = 128
_G = _B // 16
_W = 16
_WS = _W - 1
_NB = 7
_INV = 0.0078125


def _iou_conflict(bx1, by1, bx2, by2, bar, tx1, ty1, tx2, ty2, tar):
    xx1 = jnp.maximum(bx1, tx1)
    yy1 = jnp.maximum(by1, ty1)
    xx2 = jnp.minimum(bx2, tx2)
    yy2 = jnp.minimum(by2, ty2)
    w = jnp.maximum(0.0, xx2 - xx1)
    h = jnp.maximum(0.0, yy2 - yy1)
    inter = w * h
    iou = inter / ((bar + tar) - inter + 1e-8)
    return iou > _THR


def _make_sc_nms(npad):
    nblk = npad // _B
    capb = ((npad // _WS) + 31) & ~15
    f32, i32 = jnp.float32, jnp.int32
    mesh = plsc.VectorSubcoreMesh(
        core_axis_name="c", subcore_axis_name="s", num_cores=1
    )

    @functools.partial(
        pl.kernel,
        mesh=mesh,
        out_type=jax.ShapeDtypeStruct((npad,), jnp.float32),
        compiler_params=pltpu.CompilerParams(needs_layout_passes=False),
        interpret=_INTERPRET,
        scratch_types=[
            pltpu.VMEM((npad,), f32),
            pltpu.VMEM((npad,), f32),
            pltpu.VMEM((npad,), f32),
            pltpu.VMEM((npad,), f32),
            pltpu.VMEM((_NB * capb + 16,), i32),
            pltpu.VMEM((_B,), f32),
            pltpu.VMEM((_B,), f32),
            pltpu.VMEM((_W, _B), f32),
            pltpu.VMEM((_B + 32,), f32),
            pltpu.VMEM((_B + 32,), f32),
            pltpu.VMEM((_B + 32,), f32),
            pltpu.VMEM((_B + 32,), f32),
            pltpu.VMEM((_B + 32,), f32),
            pltpu.VMEM((_B + 32,), i32),
            pltpu.VMEM((_B + 32,), i32),
            pltpu.VMEM_SHARED((_W, _B), f32),
            pltpu.VMEM_SHARED((_B,), f32),
        ],
    )
    def sc_nms(x1h, y1h, x2h, y2h, keep_h, vx1, vy1, vx2, vy2,
               binl, mymask, av, slots_l, tsx1, tsy1, tsx2, tsy2, tsar,
               tsgx, tsol, slots_s, alive_s):
        wid = lax.axis_index("s")
        iota16 = lax.broadcasted_iota(i32, (16,), 0)
        zeros16 = jnp.zeros((16,), f32)

        pltpu.sync_copy(x1h, vx1)
        pltpu.sync_copy(y1h, vy1)
        pltpu.sync_copy(x2h, vx2)
        pltpu.sync_copy(y2h, vy2)

        def stage_targets(tbase):
            tcnt = [jnp.int32(0)] * _NB
            for g in range(_G):
                x1v = vx1[pl.ds(tbase + g * 16, 16)]
                gxv = (x1v * _INV).astype(i32)
                for b in range(_NB):
                    tcnt[b] = tcnt[b] + jnp.sum((gxv == b).astype(i32))
            rb = [jnp.int32(0)] * _NB
            run = jnp.int32(0)
            for b in range(_NB):
                rb[b] = run
                run = run + tcnt[b]
            for g in range(_G):
                x1v = vx1[pl.ds(tbase + g * 16, 16)]
                y1v = vy1[pl.ds(tbase + g * 16, 16)]
                x2v = vx2[pl.ds(tbase + g * 16, 16)]
                y2v = vy2[pl.ds(tbase + g * 16, 16)]
                arv = (x2v - x1v) * (y2v - y1v)
                gxv = (x1v * _INV).astype(i32)
                olv = g * 16 + iota16
                for b in range(_NB):
                    mb = gxv == b
                    mi = mb.astype(i32)
                    pos = rb[b] + (jnp.cumsum(mi) - mi)
                    plsc.store_scatter(tsx1, [pos], x1v, mask=mb)
                    plsc.store_scatter(tsy1, [pos], y1v, mask=mb)
                    plsc.store_scatter(tsx2, [pos], x2v, mask=mb)
                    plsc.store_scatter(tsy2, [pos], y2v, mask=mb)
                    plsc.store_scatter(tsar, [pos], arv, mask=mb)
                    plsc.store_scatter(tsgx, [pos], gxv, mask=mb)
                    plsc.store_scatter(tsol, [pos], olv, mask=mb)
                    rb[b] = rb[b] + jnp.sum(mi)

        def cross_binned(lo_bc, hi_bc, accumulate):
            def sgbody(sg, _):
                t0 = sg * 16
                stx1 = tsx1[pl.ds(t0, 16)]
                sty1 = tsy1[pl.ds(t0, 16)]
                stx2 = tsx2[pl.ds(t0, 16)]
                sty2 = tsy2[pl.ds(t0, 16)]
                star = tsar[pl.ds(t0, 16)]
                tol = tsol[pl.ds(t0, 16)]
                glo = tsgx[pl.ds(t0, 16)][0]
                ghi = tsgx[pl.ds(t0 + 15, 16)][0]
                acc = zeros16
                for b in range(_NB):
                    inb = (b >= glo - 1) & (b <= ghi + 1)
                    lo = jnp.where(inb, lo_bc[b], 0)
                    hi = jnp.where(inb, hi_bc[b], 0)

                    def sb(s, a, _b=b):
                        iv = plsc.load_gather(
                            binl, [jnp.full((16,), _b * capb + s, i32)]
                        )
                        bx1 = plsc.load_gather(vx1, [iv])
                        by1 = plsc.load_gather(vy1, [iv])
                        bx2 = plsc.load_gather(vx2, [iv])
                        by2 = plsc.load_gather(vy2, [iv])
                        bar = (bx2 - bx1) * (by2 - by1)
                        conf = _iou_conflict(bx1, by1, bx2, by2, bar,
                                             stx1, sty1, stx2, sty2, star)
                        return jnp.where(conf, 1.0, a)

                    acc = lax.fori_loop(lo, hi, sb, acc)
                if accumulate:
                    old = plsc.load_gather(mymask, [tol])
                    acc = jnp.maximum(old, acc)
                plsc.store_scatter(mymask, [tol], acc)
                return 0

            lax.fori_loop(0, _G, sgbody, 0)

        plsc.subcore_barrier()

        @pl.when(wid == 0)
        def _zero_slot0():
            for g in range(_G):
                mymask[pl.ds(g * 16, 16)] = zeros16
            pltpu.sync_copy(mymask, slots_s.at[0])

        @pl.when(wid > 0)
        def _zero_mask():
            for g in range(_G):
                mymask[pl.ds(g * 16, 16)] = zeros16
            stage_targets(0)

        plsc.subcore_barrier()

        def loop(k, carry):
            gcnt = carry[0]
            obc = list(carry[1:])
            base = k * _B

            nbc = list(obc)
            ngc = gcnt
            for g in range(_G):
                a = av[pl.ds(g * 16, 16)]
                live = (a > 0.5) & (k > 0)
                ai = live.astype(i32)
                inc = jnp.cumsum(ai)
                ordv = ngc + (inc - ai)
                mine = live & ((ordv % _WS) == (wid - 1))
                gidx = (base - _B) + g * 16 + iota16
                x1v = vx1[pl.ds(jnp.maximum(base - _B, 0) + g * 16, 16)]
                gxv = (x1v * _INV).astype(i32)
                for b in range(_NB):
                    mb = mine & (gxv == b)
                    mi = mb.astype(i32)
                    pos = nbc[b] + (jnp.cumsum(mi) - mi) + b * capb
                    plsc.store_scatter(binl, [pos], gidx, mask=mb)
                    nbc[b] = nbc[b] + jnp.sum(mi)
                ngc = ngc + jnp.sum(ai)

            @pl.when((wid > 0) & (k > 0))
            def _():
                cross_binned(obc, nbc, accumulate=True)

            @pl.when(wid > 0)
            def _():
                pltpu.sync_copy(mymask, slots_s.at[wid])

            plsc.subcore_barrier()

            @pl.when(wid == 0)
            def _resolve():
                pltpu.sync_copy(slots_s, slots_l)
                for g in range(_G):
                    acc = zeros16
                    for w_ in range(_W):
                        acc = jnp.maximum(acc, slots_l[w_, pl.ds(g * 16, 16)])
                    av[pl.ds(g * 16, 16)] = 1.0 - acc

                def rbody(i, _2):
                    a_i = plsc.load_gather(av, [jnp.full((16,), i, i32)])[0]

                    @pl.when(a_i > 0.5)
                    def _3():
                        giv = jnp.full((16,), base + i, i32)
                        bx1 = plsc.load_gather(vx1, [giv])
                        by1 = plsc.load_gather(vy1, [giv])
                        bx2 = plsc.load_gather(vx2, [giv])
                        by2 = plsc.load_gather(vy2, [giv])
                        bar = (bx2 - bx1) * (by2 - by1)

                        def gbody(g, _4):
                            toff = base + g * 16
                            tx1 = vx1[pl.ds(toff, 16)]
                            ty1 = vy1[pl.ds(toff, 16)]
                            tx2 = vx2[pl.ds(toff, 16)]
                            ty2 = vy2[pl.ds(toff, 16)]
                            tar = (tx2 - tx1) * (ty2 - ty1)
                            conf = _iou_conflict(bx1, by1, bx2, by2, bar,
                                                 tx1, ty1, tx2, ty2, tar)
                            conf = conf & ((g * 16 + iota16) > i)
                            cur = av[pl.ds(g * 16, 16)]
                            av[pl.ds(g * 16, 16)] = jnp.where(conf, 0.0, cur)
                            return 0

                        lax.fori_loop(i // 16, _G, gbody, 0)
                    return 0

                lax.fori_loop(0, _B, rbody, 0)
                pltpu.sync_copy(av, alive_s)
                pltpu.sync_copy(av, keep_h.at[pl.ds(base, _B)])

            @pl.when((wid > 0) & (k + 1 < nblk))
            def _():
                stage_targets(base + _B)
                zero = [jnp.int32(0)] * _NB
                cross_binned(zero, nbc, accumulate=False)

            plsc.subcore_barrier()
            pltpu.sync_copy(alive_s, av)
            return tuple([ngc] + nbc)

        lax.fori_loop(0, nblk, loop,
                      tuple([jnp.int32(0)] * (1 + _NB)))

    return sc_nms


@jax.jit
def kernel(boxes, scores):
    n = boxes.shape[0]
    order = jnp.argsort(-scores)
    b = jnp.take(boxes, order, axis=0)
    s = jnp.take(scores, order)

    nblk = (n + _B - 1) // _B
    npad = nblk * _B
    bp = jnp.pad(b, ((0, npad - n), (0, 0)))
    keep = _make_sc_nms(npad)(
        bp[:, 0], bp[:, 1], bp[:, 2], bp[:, 3]
    )[:n]
    return jnp.concatenate([b * keep[:, None], (s * keep)[:, None]], axis=1)


# speedup vs baseline: 1.2538x; 1.2538x over previous
import functools

import jax
import jax.numpy as jnp
from jax import lax
from jax.experimental import pallas as pl
from jax.experimental.pallas import tpu as pltpu
from jax.experimental.pallas import tpu_sc as plsc

_THR = 0.3
_INTERPRET = False
_B = 128
_G = _B // 16
_W = 16
_WS = _W - 1


def _iou_conflict(bx1, by1, bx2, by2, bar, tx1, ty1, tx2, ty2, tar):
    xx1 = jnp.maximum(bx1, tx1)
    yy1 = jnp.maximum(by1, ty1)
    xx2 = jnp.minimum(bx2, tx2)
    yy2 = jnp.minimum(by2, ty2)
    w = jnp.maximum(0.0, xx2 - xx1)
    h = jnp.maximum(0.0, yy2 - yy1)
    inter = w * h
    iou = inter / ((bar + tar) - inter + 1e-8)
    return iou > _THR


def _make_sc_nms(npad):
    nblk = npad // _B
    cap = ((npad // _WS) + 31) & ~15
    f32, i32 = jnp.float32, jnp.int32
    mesh = plsc.VectorSubcoreMesh(
        core_axis_name="c", subcore_axis_name="s", num_cores=1
    )

    @functools.partial(
        pl.kernel,
        mesh=mesh,
        out_type=jax.ShapeDtypeStruct((npad,), jnp.float32),
        compiler_params=pltpu.CompilerParams(needs_layout_passes=False),
        interpret=_INTERPRET,
        scratch_types=[
            pltpu.VMEM((npad,), f32),
            pltpu.VMEM((npad,), f32),
            pltpu.VMEM((npad,), f32),
            pltpu.VMEM((npad,), f32),
            pltpu.VMEM((cap,), i32),
            pltpu.VMEM((_B,), f32),
            pltpu.VMEM((_B,), f32),
            pltpu.VMEM((_W, _B), f32),
            pltpu.VMEM_SHARED((_W, _B), f32),
            pltpu.VMEM_SHARED((_B,), f32),
        ],
    )
    def sc_nms(x1h, y1h, x2h, y2h, keep_h, vx1, vy1, vx2, vy2,
               surv, mymask, av, slots_l, slots_s, alive_s):
        wid = lax.axis_index("s")
        iota16 = lax.broadcasted_iota(i32, (16,), 0)
        zeros16 = jnp.zeros((16,), f32)

        pltpu.sync_copy(x1h, vx1)
        pltpu.sync_copy(y1h, vy1)
        pltpu.sync_copy(x2h, vx2)
        pltpu.sync_copy(y2h, vy2)

        def cross_range(tbase, lo, hi):
            for half in range(2):
                toff = tbase + half * 64
                tx1 = [vx1[pl.ds(toff + g * 16, 16)] for g in range(4)]
                ty1 = [vy1[pl.ds(toff + g * 16, 16)] for g in range(4)]
                tx2 = [vx2[pl.ds(toff + g * 16, 16)] for g in range(4)]
                ty2 = [vy2[pl.ds(toff + g * 16, 16)] for g in range(4)]
                tar = [(tx2[g] - tx1[g]) * (ty2[g] - ty1[g]) for g in range(4)]

                def one(s, accs, _tx1=tx1, _ty1=ty1, _tx2=tx2, _ty2=ty2,
                        _tar=tar):
                    iv = plsc.load_gather(surv, [jnp.full((16,), s, i32)])
                    bx1 = plsc.load_gather(vx1, [iv])
                    by1 = plsc.load_gather(vy1, [iv])
                    bx2 = plsc.load_gather(vx2, [iv])
                    by2 = plsc.load_gather(vy2, [iv])
                    bar = (bx2 - bx1) * (by2 - by1)
                    out = []
                    for g in range(4):
                        conf = _iou_conflict(bx1, by1, bx2, by2, bar,
                                             _tx1[g], _ty1[g], _tx2[g],
                                             _ty2[g], _tar[g])
                        out.append(jnp.where(conf, 1.0, accs[g]))
                    return tuple(out)

                def two(t, accs):
                    return one(lo + 2 * t + 1, one(lo + 2 * t, accs))

                init = tuple(
                    mymask[pl.ds(half * 64 + g * 16, 16)] for g in range(4)
                )
                npairs = (hi - lo) // 2
                accs = lax.fori_loop(0, npairs, two, init)
                accs = lax.cond(
                    lo + 2 * npairs < hi,
                    lambda a: one(hi - 1, a),
                    lambda a: a,
                    accs,
                )
                for g in range(4):
                    mymask[pl.ds(half * 64 + g * 16, 16)] = accs[g]

        def my_shard_count(gc):
            return jnp.maximum(0, (gc - (wid - 1) + (_WS - 1)) // _WS)

        @pl.when(wid == 0)
        def _zero_slot0():
            for g in range(_G):
                mymask[pl.ds(g * 16, 16)] = zeros16
            pltpu.sync_copy(mymask, slots_s.at[0])

        @pl.when(wid > 0)
        def _zero_mask():
            for g in range(_G):
                mymask[pl.ds(g * 16, 16)] = zeros16

        plsc.subcore_barrier()

        def loop(k, gcnt):
            base = k * _B

            omy = my_shard_count(gcnt)

            def abody(g, gc):
                a = av[pl.ds(g * 16, 16)]
                ai = a.astype(i32)
                inc = jnp.cumsum(ai)
                ordv = gc + (inc - ai)
                mine = (a > 0.5) & ((ordv % _WS) == (wid - 1))
                pos = ordv // _WS
                gidx = (base - _B) + g * 16 + iota16
                plsc.store_scatter(surv, [pos], gidx, mask=mine)
                return gc + jnp.sum(ai)

            @pl.when((wid > 0) & (k > 0))
            def _():
                lax.fori_loop(0, _G, abody, gcnt)

            def cbody(g, gc):
                return gc + jnp.sum(av[pl.ds(g * 16, 16)].astype(i32))

            gcnt2 = lax.cond(
                k > 0,
                lambda: lax.fori_loop(0, _G, cbody, gcnt),
                lambda: gcnt,
            )
            nmy = my_shard_count(gcnt2)

            @pl.when((wid > 0) & (k > 0))
            def _():
                cross_range(base, omy, nmy)

            @pl.when(wid > 0)
            def _():
                pltpu.sync_copy(mymask, slots_s.at[wid])

            plsc.subcore_barrier()

            @pl.when(wid == 0)
            def _resolve():
                pltpu.sync_copy(slots_s, slots_l)
                for g in range(_G):
                    acc = zeros16
                    for w_ in range(_W):
                        acc = jnp.maximum(acc, slots_l[w_, pl.ds(g * 16, 16)])
                    av[pl.ds(g * 16, 16)] = 1.0 - acc

                def rbody(i, _2):
                    a_i = plsc.load_gather(av, [jnp.full((16,), i, i32)])[0]

                    @pl.when(a_i > 0.5)
                    def _3():
                        giv = jnp.full((16,), base + i, i32)
                        bx1 = plsc.load_gather(vx1, [giv])
                        by1 = plsc.load_gather(vy1, [giv])
                        bx2 = plsc.load_gather(vx2, [giv])
                        by2 = plsc.load_gather(vy2, [giv])
                        bar = (bx2 - bx1) * (by2 - by1)

                        def gbody(g, _4):
                            toff = base + g * 16
                            tx1 = vx1[pl.ds(toff, 16)]
                            ty1 = vy1[pl.ds(toff, 16)]
                            tx2 = vx2[pl.ds(toff, 16)]
                            ty2 = vy2[pl.ds(toff, 16)]
                            tar = (tx2 - tx1) * (ty2 - ty1)
                            conf = _iou_conflict(bx1, by1, bx2, by2, bar,
                                                 tx1, ty1, tx2, ty2, tar)
                            conf = conf & ((g * 16 + iota16) > i)
                            cur = av[pl.ds(g * 16, 16)]
                            av[pl.ds(g * 16, 16)] = jnp.where(conf, 0.0, cur)
                            return 0

                        lax.fori_loop(i // 16, _G, gbody, 0)
                    return 0

                lax.fori_loop(0, _B, rbody, 0)
                pltpu.sync_copy(av, alive_s)
                pltpu.sync_copy(av, keep_h.at[pl.ds(base, _B)])

            @pl.when((wid > 0) & (k + 1 < nblk))
            def _():
                for g in range(_G):
                    mymask[pl.ds(g * 16, 16)] = zeros16
                cross_range(base + _B, 0, nmy)

            plsc.subcore_barrier()

            pltpu.sync_copy(alive_s, av)
            return gcnt2

        lax.fori_loop(0, nblk, loop, jnp.int32(0))

    return sc_nms


@jax.jit
def kernel(boxes, scores):
    n = boxes.shape[0]
    order = jnp.argsort(-scores)
    b = jnp.take(boxes, order, axis=0)
    s = jnp.take(scores, order)

    nblk = (n + _B - 1) // _B
    npad = nblk * _B
    bp = jnp.pad(b, ((0, npad - n), (0, 0)))
    keep = _make_sc_nms(npad)(
        bp[:, 0], bp[:, 1], bp[:, 2], bp[:, 3]
    )[:n]
    return jnp.concatenate([b * keep[:, None], (s * keep)[:, None]], axis=1)


# speedup vs baseline: 1.2714x; 1.0141x over previous
import functools

import jax
import jax.numpy as jnp
from jax import lax
from jax.experimental import pallas as pl
from jax.experimental.pallas import tpu as pltpu
from jax.experimental.pallas import tpu_sc as plsc

_THR = 0.3
_INTERPRET = False
_B = 128
_G = _B // 16
_W = 16
_WS = _W - 1


def _iou_conflict(bx1, by1, bx2, by2, bar, tx1, ty1, tx2, ty2, tar):
    xx1 = jnp.maximum(bx1, tx1)
    yy1 = jnp.maximum(by1, ty1)
    xx2 = jnp.minimum(bx2, tx2)
    yy2 = jnp.minimum(by2, ty2)
    w = jnp.maximum(0.0, xx2 - xx1)
    h = jnp.maximum(0.0, yy2 - yy1)
    inter = w * h
    iou = inter / ((bar + tar) - inter + 1e-8)
    return iou > _THR


def _make_sc_nms(npad):
    nblk = npad // _B
    cap = ((npad // _WS) + 31) & ~15
    f32, i32 = jnp.float32, jnp.int32
    mesh = plsc.VectorSubcoreMesh(
        core_axis_name="c", subcore_axis_name="s", num_cores=1
    )

    @functools.partial(
        pl.kernel,
        mesh=mesh,
        out_type=jax.ShapeDtypeStruct((npad,), jnp.float32),
        compiler_params=pltpu.CompilerParams(needs_layout_passes=False),
        interpret=_INTERPRET,
        scratch_types=[
            pltpu.VMEM((npad,), f32),
            pltpu.VMEM((npad,), f32),
            pltpu.VMEM((npad,), f32),
            pltpu.VMEM((npad,), f32),
            pltpu.VMEM((cap,), i32),
            pltpu.VMEM((_B,), f32),
            pltpu.VMEM((_B,), f32),
            pltpu.VMEM((_W, _B), f32),
            pltpu.VMEM_SHARED((_W, _B), f32),
            pltpu.VMEM_SHARED((_B,), f32),
        ],
    )
    def sc_nms(x1h, y1h, x2h, y2h, keep_h, vx1, vy1, vx2, vy2,
               surv, mymask, av, slots_l, slots_s, alive_s):
        wid = lax.axis_index("s")
        iota16 = lax.broadcasted_iota(i32, (16,), 0)
        zeros16 = jnp.zeros((16,), f32)

        pltpu.sync_copy(x1h, vx1)
        pltpu.sync_copy(y1h, vy1)
        pltpu.sync_copy(x2h, vx2)
        pltpu.sync_copy(y2h, vy2)

        def cross_range(tbase, lo, hi):
            for half in range(2):
                toff = tbase + half * 64
                tx1 = [vx1[pl.ds(toff + g * 16, 16)] for g in range(4)]
                ty1 = [vy1[pl.ds(toff + g * 16, 16)] for g in range(4)]
                tx2 = [vx2[pl.ds(toff + g * 16, 16)] for g in range(4)]
                ty2 = [vy2[pl.ds(toff + g * 16, 16)] for g in range(4)]
                tar = [(tx2[g] - tx1[g]) * (ty2[g] - ty1[g]) for g in range(4)]

                def sbody(s, accs, _tx1=tx1, _ty1=ty1, _tx2=tx2, _ty2=ty2,
                          _tar=tar):
                    iv = plsc.load_gather(surv, [jnp.full((16,), s, i32)])
                    bx1 = plsc.load_gather(vx1, [iv])
                    by1 = plsc.load_gather(vy1, [iv])
                    bx2 = plsc.load_gather(vx2, [iv])
                    by2 = plsc.load_gather(vy2, [iv])
                    bar = (bx2 - bx1) * (by2 - by1)
                    out = []
                    for g in range(4):
                        conf = _iou_conflict(bx1, by1, bx2, by2, bar,
                                             _tx1[g], _ty1[g], _tx2[g],
                                             _ty2[g], _tar[g])
                        out.append(jnp.where(conf, 1.0, accs[g]))
                    return tuple(out)

                init = tuple(
                    mymask[pl.ds(half * 64 + g * 16, 16)] for g in range(4)
                )
                accs = lax.fori_loop(lo, hi, sbody, init)
                for g in range(4):
                    mymask[pl.ds(half * 64 + g * 16, 16)] = accs[g]

        def my_shard_count(gc):
            return jnp.maximum(0, (gc - (wid - 1) + (_WS - 1)) // _WS)

        @pl.when(wid == 0)
        def _zero_slot0():
            for g in range(_G):
                mymask[pl.ds(g * 16, 16)] = zeros16
            pltpu.sync_copy(mymask, slots_s.at[0])

        @pl.when(wid > 0)
        def _zero_mask():
            for g in range(_G):
                mymask[pl.ds(g * 16, 16)] = zeros16

        plsc.subcore_barrier()

        def loop(k, gcnt):
            base = k * _B

            omy = my_shard_count(gcnt)

            def abody(g, gc):
                a = av[pl.ds(g * 16, 16)]
                ai = a.astype(i32)
                inc = jnp.cumsum(ai)
                ordv = gc + (inc - ai)
                mine = (a > 0.5) & ((ordv % _WS) == (wid - 1))
                pos = ordv // _WS
                gidx = (base - _B) + g * 16 + iota16
                plsc.store_scatter(surv, [pos], gidx, mask=mine)
                return gc + jnp.sum(ai)

            @pl.when((wid > 0) & (k > 0))
            def _():
                lax.fori_loop(0, _G, abody, gcnt)

            def cbody(g, gc):
                return gc + jnp.sum(av[pl.ds(g * 16, 16)].astype(i32))

            gcnt2 = lax.cond(
                k > 0,
                lambda: lax.fori_loop(0, _G, cbody, gcnt),
                lambda: gcnt,
            )
            nmy = my_shard_count(gcnt2)

            @pl.when((wid > 0) & (k > 0))
            def _():
                cross_range(base, omy, nmy)

            @pl.when(wid > 0)
            def _():
                pltpu.sync_copy(mymask, slots_s.at[wid])

            plsc.subcore_barrier()

            @pl.when(wid == 0)
            def _resolve():
                pltpu.sync_copy(slots_s, slots_l)
                for g in range(_G):
                    acc = zeros16
                    for w_ in range(_W):
                        acc = jnp.maximum(acc, slots_l[w_, pl.ds(g * 16, 16)])
                    av[pl.ds(g * 16, 16)] = 1.0 - acc

                def rbody(i, _2):
                    a_i = plsc.load_gather(av, [jnp.full((16,), i, i32)])[0]

                    @pl.when(a_i > 0.5)
                    def _3():
                        giv = jnp.full((16,), base + i, i32)
                        bx1 = plsc.load_gather(vx1, [giv])
                        by1 = plsc.load_gather(vy1, [giv])
                        bx2 = plsc.load_gather(vx2, [giv])
                        by2 = plsc.load_gather(vy2, [giv])
                        bar = (bx2 - bx1) * (by2 - by1)

                        def gbody(g, _4):
                            toff = base + g * 16
                            tx1 = vx1[pl.ds(toff, 16)]
                            ty1 = vy1[pl.ds(toff, 16)]
                            tx2 = vx2[pl.ds(toff, 16)]
                            ty2 = vy2[pl.ds(toff, 16)]
                            tar = (tx2 - tx1) * (ty2 - ty1)
                            conf = _iou_conflict(bx1, by1, bx2, by2, bar,
                                                 tx1, ty1, tx2, ty2, tar)
                            conf = conf & ((g * 16 + iota16) > i)
                            cur = av[pl.ds(g * 16, 16)]
                            av[pl.ds(g * 16, 16)] = jnp.where(conf, 0.0, cur)
                            return 0

                        lax.fori_loop(i // 16, _G, gbody, 0)
                    return 0

                lax.fori_loop(0, _B, rbody, 0)
                pltpu.sync_copy(av, alive_s)
                pltpu.sync_copy(av, keep_h.at[pl.ds(base, _B)])

            @pl.when((wid > 0) & (k + 1 < nblk))
            def _():
                for g in range(_G):
                    mymask[pl.ds(g * 16, 16)] = zeros16
                cross_range(base + _B, 0, nmy)

            plsc.subcore_barrier()

            pltpu.sync_copy(alive_s, av)
            return gcnt2

        lax.fori_loop(0, nblk, loop, jnp.int32(0))

    return sc_nms


@jax.jit
def kernel(boxes, scores):
    n = boxes.shape[0]
    order = jnp.argsort(-scores)
    b = jnp.take(boxes, order, axis=0)
    s = jnp.take(scores, order)

    nblk = (n + _B - 1) // _B
    npad = nblk * _B
    bp = jnp.pad(b, ((0, npad - n), (0, 0)))
    keep = _make_sc_nms(npad)(
        bp[:, 0], bp[:, 1], bp[:, 2], bp[:, 3]
    )[:n]
    return jnp.concatenate([b * keep[:, None], (s * keep)[:, None]], axis=1)


# speedup vs baseline: 1.2715x; 1.0001x over previous
import functools

import jax
import jax.numpy as jnp
from jax import lax
from jax.experimental import pallas as pl
from jax.experimental.pallas import tpu as pltpu
from jax.experimental.pallas import tpu_sc as plsc

_THR = 0.3
_B = 128
_G = _B // 16
_W = 16
_WS = _W - 1


def _iou_conflict(bx1, by1, bx2, by2, bar, tx1, ty1, tx2, ty2, tar):
    xx1 = jnp.maximum(bx1, tx1)
    yy1 = jnp.maximum(by1, ty1)
    xx2 = jnp.minimum(bx2, tx2)
    yy2 = jnp.minimum(by2, ty2)
    w = jnp.maximum(0.0, xx2 - xx1)
    h = jnp.maximum(0.0, yy2 - yy1)
    inter = w * h
    iou = inter / ((bar + tar) - inter + 1e-8)
    return iou > _THR


def _make_sc_nms(npad):
    nblk = npad // _B
    cap = ((npad // _WS) + 31) & ~15
    f32, i32 = jnp.float32, jnp.int32
    mesh = plsc.VectorSubcoreMesh(
        core_axis_name="c", subcore_axis_name="s", num_cores=1
    )

    @functools.partial(
        pl.kernel,
        mesh=mesh,
        out_type=jax.ShapeDtypeStruct((npad,), jnp.float32),
        compiler_params=pltpu.CompilerParams(needs_layout_passes=False),
        scratch_types=[
            pltpu.VMEM((npad,), f32),
            pltpu.VMEM((npad,), f32),
            pltpu.VMEM((npad,), f32),
            pltpu.VMEM((npad,), f32),
            pltpu.VMEM((cap,), i32),
            pltpu.VMEM((_B,), f32),
            pltpu.VMEM((_B,), f32),
            pltpu.VMEM((_W, _B), f32),
            pltpu.VMEM_SHARED((_W, _B), f32),
            pltpu.VMEM_SHARED((_B,), f32),
        ],
    )
    def sc_nms(x1h, y1h, x2h, y2h, keep_h, vx1, vy1, vx2, vy2,
               surv, mymask, av, slots_l, slots_s, alive_s):
        wid = lax.axis_index("s")
        iota16 = lax.broadcasted_iota(i32, (16,), 0)
        zeros16 = jnp.zeros((16,), f32)

        pltpu.sync_copy(x1h, vx1)
        pltpu.sync_copy(y1h, vy1)
        pltpu.sync_copy(x2h, vx2)
        pltpu.sync_copy(y2h, vy2)

        def cross_range(tbase, lo, hi):
            for half in range(2):
                toff = tbase + half * 64
                tx1 = [vx1[pl.ds(toff + g * 16, 16)] for g in range(4)]
                ty1 = [vy1[pl.ds(toff + g * 16, 16)] for g in range(4)]
                tx2 = [vx2[pl.ds(toff + g * 16, 16)] for g in range(4)]
                ty2 = [vy2[pl.ds(toff + g * 16, 16)] for g in range(4)]
                tar = [(tx2[g] - tx1[g]) * (ty2[g] - ty1[g]) for g in range(4)]

                def sbody(s, accs, _tx1=tx1, _ty1=ty1, _tx2=tx2, _ty2=ty2,
                          _tar=tar):
                    iv = plsc.load_gather(surv, [jnp.full((16,), s, i32)])
                    bx1 = plsc.load_gather(vx1, [iv])
                    by1 = plsc.load_gather(vy1, [iv])
                    bx2 = plsc.load_gather(vx2, [iv])
                    by2 = plsc.load_gather(vy2, [iv])
                    bar = (bx2 - bx1) * (by2 - by1)
                    out = []
                    for g in range(4):
                        conf = _iou_conflict(bx1, by1, bx2, by2, bar,
                                             _tx1[g], _ty1[g], _tx2[g],
                                             _ty2[g], _tar[g])
                        out.append(jnp.where(conf, 1.0, accs[g]))
                    return tuple(out)

                init = tuple(
                    mymask[pl.ds(half * 64 + g * 16, 16)] for g in range(4)
                )
                accs = lax.fori_loop(lo, hi, sbody, init)
                for g in range(4):
                    mymask[pl.ds(half * 64 + g * 16, 16)] = accs[g]

        def my_shard_count(gc):
            return jnp.maximum(0, (gc - (wid - 1) + (_WS - 1)) // _WS)

        @pl.when(wid == 0)
        def _zero_slot0():
            for g in range(_G):
                mymask[pl.ds(g * 16, 16)] = zeros16
            pltpu.sync_copy(mymask, slots_s.at[0])

        @pl.when(wid > 0)
        def _zero_mask():
            for g in range(_G):
                mymask[pl.ds(g * 16, 16)] = zeros16

        plsc.subcore_barrier()

        def loop(k, gcnt):
            base = k * _B

            omy = my_shard_count(gcnt)

            def abody(g, gc):
                a = av[pl.ds(g * 16, 16)]
                ai = a.astype(i32)
                inc = jnp.cumsum(ai)
                ordv = gc + (inc - ai)
                mine = (a > 0.5) & ((ordv % _WS) == (wid - 1))
                pos = ordv // _WS
                gidx = (base - _B) + g * 16 + iota16
                plsc.store_scatter(surv, [pos], gidx, mask=mine)
                return gc + jnp.sum(ai)

            @pl.when((wid > 0) & (k > 0))
            def _():
                lax.fori_loop(0, _G, abody, gcnt)

            def cbody(g, gc):
                return gc + jnp.sum(av[pl.ds(g * 16, 16)].astype(i32))

            gcnt2 = lax.cond(
                k > 0,
                lambda: lax.fori_loop(0, _G, cbody, gcnt),
                lambda: gcnt,
            )
            nmy = my_shard_count(gcnt2)

            @pl.when((wid > 0) & (k > 0))
            def _():
                cross_range(base, omy, nmy)

            @pl.when(wid > 0)
            def _():
                pltpu.sync_copy(mymask, slots_s.at[wid])

            plsc.subcore_barrier()

            @pl.when(wid == 0)
            def _resolve():
                pltpu.sync_copy(slots_s, slots_l)
                for g in range(_G):
                    acc = zeros16
                    for w_ in range(_W):
                        acc = jnp.maximum(acc, slots_l[w_, pl.ds(g * 16, 16)])
                    av[pl.ds(g * 16, 16)] = 1.0 - acc

                def rbody(i, _2):
                    a_i = plsc.load_gather(av, [jnp.full((16,), i, i32)])[0]

                    @pl.when(a_i > 0.5)
                    def _3():
                        giv = jnp.full((16,), base + i, i32)
                        bx1 = plsc.load_gather(vx1, [giv])
                        by1 = plsc.load_gather(vy1, [giv])
                        bx2 = plsc.load_gather(vx2, [giv])
                        by2 = plsc.load_gather(vy2, [giv])
                        bar = (bx2 - bx1) * (by2 - by1)

                        def gbody(g, _4):
                            toff = base + g * 16
                            tx1 = vx1[pl.ds(toff, 16)]
                            ty1 = vy1[pl.ds(toff, 16)]
                            tx2 = vx2[pl.ds(toff, 16)]
                            ty2 = vy2[pl.ds(toff, 16)]
                            tar = (tx2 - tx1) * (ty2 - ty1)
                            conf = _iou_conflict(bx1, by1, bx2, by2, bar,
                                                 tx1, ty1, tx2, ty2, tar)
                            conf = conf & ((g * 16 + iota16) > i)
                            cur = av[pl.ds(g * 16, 16)]
                            av[pl.ds(g * 16, 16)] = jnp.where(conf, 0.0, cur)
                            return 0

                        lax.fori_loop(i // 16, _G, gbody, 0)
                    return 0

                lax.fori_loop(0, _B, rbody, 0)
                pltpu.sync_copy(av, alive_s)
                pltpu.sync_copy(av, keep_h.at[pl.ds(base, _B)])

            @pl.when((wid > 0) & (k + 1 < nblk))
            def _():
                for g in range(_G):
                    mymask[pl.ds(g * 16, 16)] = zeros16
                cross_range(base + _B, 0, nmy)

            plsc.subcore_barrier()

            pltpu.sync_copy(alive_s, av)
            return gcnt2

        lax.fori_loop(0, nblk, loop, jnp.int32(0))

    return sc_nms


@jax.jit
def kernel(boxes, scores):
    n = boxes.shape[0]
    order = jnp.argsort(-scores)
    b = jnp.take(boxes, order, axis=0)
    s = jnp.take(scores, order)

    nblk = (n + _B - 1) // _B
    npad = nblk * _B
    bp = jnp.pad(b, ((0, npad - n), (0, 0)))
    keep = _make_sc_nms(npad)(
        bp[:, 0], bp[:, 1], bp[:, 2], bp[:, 3]
    )[:n]
    return jnp.concatenate([b * keep[:, None], (s * keep)[:, None]], axis=1)
